# baseline (speedup 1.0000x reference)
import jax
import jax.numpy as jnp
from jax import lax
from jax.experimental import pallas as pl
from jax.experimental.pallas import tpu as pltpu, tpu_sc as plsc

N = 100000
C = 128
NC, NS, L = 2, 16, 16
NW = NC * NS
B0 = 800
NBLK = N // B0
REM = NBLK % NW
NFULL = NBLK // NW + 1
G = B0 // L


def _body(inp_hbm, zblk_hbm, out_hbm, idx_v, buf):
    c = lax.axis_index("c")
    s = lax.axis_index("s")
    wid = s * NC + c

    pltpu.sync_copy(zblk_hbm, buf)

    iota = lax.iota(jnp.int32, 16)
    ones = jnp.ones((16,), jnp.int32)
    zeros = jnp.zeros((16,), jnp.int32)
    nblk = jnp.where(wid < REM, NFULL, NFULL - 1)

    def do_block(i, carry):
        blk = wid + i * NW
        rowbase = blk * B0
        pltpu.sync_copy(inp_hbm.at[pl.ds(rowbase, B0)], idx_v)

        def scat_ones(g, cc):
            vals = idx_v[pl.ds(g * L, L)]
            lin = (g * L + iota) * C + vals
            plsc.store_scatter(buf, [lin], ones)
            return cc

        def scat_zeros(g, cc):
            vals = idx_v[pl.ds(g * L, L)]
            lin = (g * L + iota) * C + vals
            plsc.store_scatter(buf, [lin], zeros)
            return cc

        lax.fori_loop(0, G, scat_ones, 0)
        pltpu.sync_copy(buf, out_hbm.at[pl.ds(rowbase * C, B0 * C)])
        lax.fori_loop(0, G, scat_zeros, 0)
        return carry

    lax.fori_loop(0, nblk, do_block, 0)


_onehot_sc = pl.kernel(
    _body,
    out_type=jax.ShapeDtypeStruct((N * C,), jnp.int32),
    mesh=plsc.VectorSubcoreMesh(core_axis_name="c", subcore_axis_name="s"),
    scratch_types=[
        pltpu.VMEM((B0,), jnp.int32),
        pltpu.VMEM((B0 * C,), jnp.int32),
    ],
    compiler_params=pltpu.CompilerParams(needs_layout_passes=False),
)


def kernel(inp):
    zblk = jnp.zeros((B0 * C,), jnp.int32)
    out = _onehot_sc(inp, zblk)
    return out.reshape(N, C)
